# baseline (speedup 1.0000x reference)
import functools
import math

import jax
import jax.numpy as jnp
import numpy as np
from jax import lax
from jax.experimental import pallas as pl
from jax.experimental.pallas import tpu as pltpu
from jax.experimental.pallas import tpu_sc as plsc

_NUM_WORKERS = 32
_LANES = 16


def _positional_encoding(maxlen, dim):
    pos = jnp.arange(maxlen, dtype=jnp.float32)
    i = np.arange(dim)
    terms = jnp.asarray(1.0 / (10000.0 ** (2.0 * (i // 2) / float(dim))),
                        dtype=jnp.float32)
    pe_val = pos[:, None] * terms[None, :]
    pe = jnp.zeros((maxlen, dim), dtype=jnp.float32)
    pe = pe.at[:, 0::2].set(jnp.sin(pe_val[:, 0::2]))
    pe = pe.at[:, 1::2].set(jnp.cos(pe_val[:, 0::2]))
    return pe


@functools.partial(jax.jit, static_argnames=("b", "l"))
def _run(idx, W, pe, b, l):
    n_chunks, chunk = idx.shape[1], idx.shape[2]
    d = W.shape[1]
    scale = 1.0 / math.sqrt(float(b))
    vregs_per_row = d // _LANES
    mesh = plsc.VectorSubcoreMesh(core_axis_name="c", subcore_axis_name="s")

    @functools.partial(
        pl.kernel,
        mesh=mesh,
        out_type=jax.ShapeDtypeStruct((_NUM_WORKERS, n_chunks, chunk, d),
                                      jnp.float32),
        scratch_types=[
            pltpu.VMEM((n_chunks, chunk), jnp.int32),
            pltpu.VMEM((l, d), jnp.float32),
            pltpu.VMEM((chunk, d), jnp.float32),
            pltpu.SemaphoreType.DMA,
        ],
        compiler_params=pltpu.CompilerParams(use_tc_tiling_on_sc=False),
    )
    def sc_kernel(w_hbm, idx_hbm, pe_hbm, out_hbm, idx_v, pe_v, buf, sem):
        wid = lax.axis_index("s") * 2 + lax.axis_index("c")
        pltpu.sync_copy(idx_hbm.at[wid], idx_v)
        pltpu.sync_copy(pe_hbm, pe_v)

        def chunk_body(c, carry):
            pltpu.async_copy(w_hbm.at[idx_v.at[c]], buf, sem).wait()
            pe_row0 = lax.rem(c, 2) * chunk

            def row_body(r, carry2):
                for j in range(vregs_per_row):
                    x = buf[r, pl.ds(j * _LANES, _LANES)]
                    p = pe_v[pe_row0 + r, pl.ds(j * _LANES, _LANES)]
                    buf[r, pl.ds(j * _LANES, _LANES)] = x * scale + p
                return carry2

            lax.fori_loop(0, chunk, row_body, 0, unroll=2)
            pltpu.sync_copy(buf, out_hbm.at[wid, c])
            return carry

        lax.fori_loop(0, n_chunks, chunk_body, 0)

    return sc_kernel(W, idx, pe)


def kernel(inp, W):
    b, l = inp.shape
    d = W.shape[1]
    chunk = 100
    total = b * l
    per_w = total // _NUM_WORKERS
    n_chunks = per_w // chunk
    idx = inp.astype(jnp.int32).reshape(_NUM_WORKERS, n_chunks, chunk)
    pe = _positional_encoding(l, d)
    out = _run(idx, W, pe, b, l)
    return out.reshape(b, l, d)


# speedup vs baseline: 1.0500x; 1.0500x over previous
import functools
import math

import jax
import jax.numpy as jnp
import numpy as np
from jax import lax
from jax.experimental import pallas as pl
from jax.experimental.pallas import tpu as pltpu
from jax.experimental.pallas import tpu_sc as plsc

_NUM_WORKERS = 32
_LANES = 16
_G = 100
_NBUF = 2


def _positional_encoding(maxlen, dim):
    pos = jnp.arange(maxlen, dtype=jnp.float32)
    i = np.arange(dim)
    terms = jnp.asarray(1.0 / (10000.0 ** (2.0 * (i // 2) / float(dim))),
                        dtype=jnp.float32)
    pe_val = pos[:, None] * terms[None, :]
    pe = jnp.zeros((maxlen, dim), dtype=jnp.float32)
    pe = pe.at[:, 0::2].set(jnp.sin(pe_val[:, 0::2]))
    pe = pe.at[:, 1::2].set(jnp.cos(pe_val[:, 0::2]))
    return pe


@functools.partial(jax.jit, static_argnames=("b", "l"))
def _run(idx, W, pe, b, l):
    n_steps = idx.shape[1] // 2
    chunk = 2 * _G
    d = W.shape[1]
    scale = 1.0 / math.sqrt(float(b))
    vregs_per_row = d // _LANES
    mesh = plsc.VectorSubcoreMesh(core_axis_name="c", subcore_axis_name="s")

    @functools.partial(
        pl.kernel,
        mesh=mesh,
        out_type=jax.ShapeDtypeStruct((_NUM_WORKERS, n_steps, chunk, d),
                                      jnp.float32),
        scratch_types=[
            pltpu.VMEM((2 * n_steps, _G), jnp.int32),
            pltpu.VMEM((l, d), jnp.float32),
            pltpu.VMEM((chunk, d), jnp.float32),
            pltpu.VMEM((chunk, d), jnp.float32),
            pltpu.VMEM((chunk, d), jnp.float32),
            pltpu.VMEM((chunk, d), jnp.float32),
            pltpu.SemaphoreType.DMA,
            pltpu.SemaphoreType.DMA,
            pltpu.SemaphoreType.DMA,
            pltpu.SemaphoreType.DMA,
        ],
        compiler_params=pltpu.CompilerParams(use_tc_tiling_on_sc=False),
    )
    def sc_kernel(w_hbm, idx_hbm, pe_hbm, out_hbm,
                  idx_v, pe_v, g0, g1, o0, o1, sg0, sg1, sw0, sw1):
        wid = lax.axis_index("s") * 2 + lax.axis_index("c")
        pltpu.sync_copy(idx_hbm.at[wid], idx_v)
        pltpu.sync_copy(pe_hbm, pe_v)

        gb, ob = (g0, g1), (o0, o1)
        sg, sw = (sg0, sg1), (sw0, sw1)

        def issue_gather(s_, bi):
            pltpu.async_copy(w_hbm.at[idx_v.at[2 * s_]],
                             gb[bi].at[pl.ds(0, _G)], sg[bi])
            pltpu.async_copy(w_hbm.at[idx_v.at[2 * s_ + 1]],
                             gb[bi].at[pl.ds(_G, _G)], sg[bi])

        def wait_gather(s_, bi):
            pltpu.make_async_copy(w_hbm.at[idx_v.at[2 * s_]],
                                  gb[bi].at[pl.ds(0, _G)], sg[bi]).wait()
            pltpu.make_async_copy(w_hbm.at[idx_v.at[2 * s_ + 1]],
                                  gb[bi].at[pl.ds(_G, _G)], sg[bi]).wait()

        def issue_wb(s_, bi):
            pltpu.async_copy(ob[bi], out_hbm.at[wid, s_], sw[bi])

        def wait_wb(s_, bi):
            pltpu.make_async_copy(ob[bi], out_hbm.at[wid, s_], sw[bi]).wait()

        def compute(bi):
            def row_body(r, carry):
                for j in range(vregs_per_row):
                    sl = pl.ds(j * _LANES, _LANES)
                    ob[bi][r, sl] = gb[bi][r, sl] * scale + pe_v[r, sl]
                return carry
            lax.fori_loop(0, chunk, row_body, 0, unroll=2)

        for bi in range(_NBUF):
            issue_gather(bi, bi)
        for bi in range(_NBUF):
            wait_gather(bi, bi)
            compute(bi)
            issue_gather(bi + _NBUF, bi)
            issue_wb(bi, bi)

        def group_body(g, carry):
            for bi in range(_NBUF):
                s_ = g * _NBUF + bi
                wait_gather(s_, bi)
                wait_wb(s_ - _NBUF, bi)
                compute(bi)
                issue_gather(s_ + _NBUF, bi)
                issue_wb(s_, bi)
            return carry

        n_groups = n_steps // _NBUF
        lax.fori_loop(1, n_groups - 1, group_body, 0)

        for bi in range(_NBUF):
            s_ = (n_groups - 1) * _NBUF + bi
            wait_gather(s_, bi)
            wait_wb(s_ - _NBUF, bi)
            compute(bi)
            issue_wb(s_, bi)
        for bi in range(_NBUF):
            wait_wb((n_groups - 1) * _NBUF + bi, bi)

    return sc_kernel(W, idx, pe)


def kernel(inp, W):
    b, l = inp.shape
    d = W.shape[1]
    total = b * l
    per_w = total // _NUM_WORKERS
    idx = inp.astype(jnp.int32).reshape(_NUM_WORKERS, per_w // _G, _G)
    pe = _positional_encoding(l, d)
    out = _run(idx, W, pe, b, l)
    return out.reshape(b, l, d)
